# baseline (speedup 1.0000x reference)
import functools

import jax
import jax.numpy as jnp
from jax import lax
from jax.experimental import pallas as pl
from jax.experimental.pallas import tpu as pltpu
from jax.experimental.pallas import tpu_sc as plsc


def _vq_body(idx_ref, prec_ref, ze_ref, book_ref, prob_ref, logp_ref, fidx_ref):
    prec = prec_ref[0]
    ze = ze_ref[0]
    book = book_ref[0]
    K = book.shape[0]
    cross = jax.lax.dot_general(
        ze, book, (((1,), (1,)), ((), ())),
        preferred_element_type=jnp.float32)
    b_sq = jnp.sum(book * book, axis=1)
    t = (2.0 * prec) * cross - prec * b_sq[None, :]
    m = jnp.max(t, axis=1, keepdims=True)
    sh = t - m
    e = jnp.exp(sh)
    s = jnp.sum(e, axis=1, keepdims=True)
    prob_ref[0] = e / s
    logp_ref[0] = sh - jnp.log(s)
    am = jnp.argmax(t, axis=1)
    base = idx_ref[pl.program_id(0)] * K
    fidx_ref[0] = (am + base)[:, None]


def _make_sc_gather(n_rows, d, rows_per_w):
    mesh = plsc.VectorSubcoreMesh(core_axis_name="c", subcore_axis_name="s")

    @functools.partial(
        pl.kernel, mesh=mesh,
        out_type=jax.ShapeDtypeStruct((n_rows, d), jnp.float32),
        scratch_types=[
            pltpu.VMEM((rows_per_w,), jnp.int32),
            pltpu.VMEM((rows_per_w, d), jnp.float32),
            pltpu.SemaphoreType.DMA,
        ],
    )
    def gather_k(table_hbm, idx_hbm, out_hbm, idx_v, rows_v, sem):
        wid = lax.axis_index("s") * 2 + lax.axis_index("c")
        base = wid * rows_per_w
        pltpu.sync_copy(idx_hbm.at[pl.ds(base, rows_per_w)], idx_v)
        pltpu.async_copy(table_hbm.at[idx_v], rows_v, sem).wait()
        pltpu.sync_copy(rows_v, out_hbm.at[pl.ds(base, rows_per_w)])

    return gather_k


@jax.jit
def _vq(ze, c_logits, books, log_param_q):
    b, n, d = ze.shape
    n_books, K, _ = books.shape
    param_q = 1.0 + jnp.exp(log_param_q)
    precision_q = 0.5 / jnp.clip(param_q, 1e-10)
    idx = jnp.argmax(c_logits, axis=-1).astype(jnp.int32)
    prec_arr = jnp.reshape(precision_q.astype(jnp.float32), (1,))

    grid_spec = pltpu.PrefetchScalarGridSpec(
        num_scalar_prefetch=2,
        grid=(b,),
        in_specs=[
            pl.BlockSpec((1, n, d), lambda i, idx, prec: (i, 0, 0)),
            pl.BlockSpec((1, K, d), lambda i, idx, prec: (idx[i], 0, 0)),
        ],
        out_specs=[
            pl.BlockSpec((1, n, K), lambda i, idx, prec: (i, 0, 0)),
            pl.BlockSpec((1, n, K), lambda i, idx, prec: (i, 0, 0)),
            pl.BlockSpec((1, n, 1), lambda i, idx, prec: (i, 0, 0)),
        ],
    )
    prob, log_prob, fidx = pl.pallas_call(
        _vq_body,
        grid_spec=grid_spec,
        out_shape=[
            jax.ShapeDtypeStruct((b, n, K), jnp.float32),
            jax.ShapeDtypeStruct((b, n, K), jnp.float32),
            jax.ShapeDtypeStruct((b, n, 1), jnp.int32),
        ],
    )(idx, prec_arr, ze, books)

    n_rows = b * n
    rows_per_w = n_rows // 32
    gather_k = _make_sc_gather(n_rows, d, rows_per_w)
    zq = gather_k(jnp.reshape(books, (n_books * K, d)),
                  jnp.reshape(fidx, (n_rows,)))
    zq = jnp.reshape(zq, (b, n, d))
    return zq, precision_q, prob, log_prob


def kernel(ze, c_logits, books, log_param_q, is_train):
    del is_train
    return _vq(ze, c_logits, books, log_param_q)
